# baseline (speedup 1.0000x reference)
import functools

import jax
import jax.numpy as jnp
from jax import lax
from jax.experimental import pallas as pl
from jax.experimental.pallas import tpu as pltpu
from jax.experimental.pallas import tpu_sc as plsc

B, C, HW, P = 8, 3, 224, 16
N = (HW // P) ** 2
D = 768
D_FF = 3072
H = 12
CAP = 10000
K = 5
REMAIN = 0.1
BN = B * N

MC = 2000
NEG = -3e38
BIGI = 2**30

CAP_PAD = 10016
DUP_SLOT = 10008
ZERO_SLOT = 10012
UPD_PAD = 1792
NSC = 32
UPW = UPD_PAD // NSC
QPW = BN // NSC


def _ln(t):
    m = jnp.mean(t, axis=-1, keepdims=True)
    v = jnp.var(t, axis=-1, keepdims=True)
    return (t - m) / jnp.sqrt(v + 1e-6)


NT = 200


def _encoder_kernel(xp_ref, wp_ref, bp_ref, cp_ref, wqkv_ref, wo_ref,
                    wfc1_ref, wfc2_ref, tok_ref):
    xp = xp_ref[0]
    tokp = jnp.dot(xp, wp_ref[...], preferred_element_type=jnp.float32) + bp_ref[...]
    tok = jnp.concatenate(
        [jnp.zeros((1, D), jnp.float32), tokp, jnp.zeros((NT - 1 - N, D), jnp.float32)],
        axis=0) + cp_ref[...]
    h = _ln(tok)
    qkv = jnp.dot(h, wqkv_ref[...], preferred_element_type=jnp.float32)
    dh = D // H
    scale = 1.0 / jnp.sqrt(jnp.float32(dh))
    kmask = jnp.where(lax.broadcasted_iota(jnp.int32, (1, NT), 1) < N + 1,
                      0.0, NEG).astype(jnp.float32)
    outs = []
    for hh in range(H):
        qh = qkv[:, hh * dh:(hh + 1) * dh]
        kh = qkv[:, D + hh * dh:D + (hh + 1) * dh]
        vh = qkv[:, 2 * D + hh * dh:2 * D + (hh + 1) * dh]
        s = lax.dot_general(qh, kh, (((1,), (1,)), ((), ())),
                            preferred_element_type=jnp.float32) * scale
        s = s + kmask
        s = s - jnp.max(s, axis=1, keepdims=True)
        e = jnp.exp(s)
        a = e / jnp.sum(e, axis=1, keepdims=True)
        outs.append(jnp.dot(a, vh, preferred_element_type=jnp.float32))
    o = jnp.concatenate(outs, axis=1)
    tok = tok + jnp.dot(o, wo_ref[...], preferred_element_type=jnp.float32)
    h2 = jnp.dot(_ln(tok), wfc1_ref[...], preferred_element_type=jnp.float32)
    tok = tok + jnp.dot(jax.nn.gelu(h2), wfc2_ref[...],
                        preferred_element_type=jnp.float32)
    tok_ref[0] = tok


def _encoder(xp, W_patch, b_patch, clspos, W_qkv, W_o, W_fc1, W_fc2):
    CPP = C * P * P
    return pl.pallas_call(
        _encoder_kernel,
        grid=(B,),
        in_specs=[
            pl.BlockSpec((1, N, CPP), lambda b: (b, 0, 0)),
            pl.BlockSpec((CPP, D), lambda b: (0, 0)),
            pl.BlockSpec((1, D), lambda b: (0, 0)),
            pl.BlockSpec((NT, D), lambda b: (0, 0)),
            pl.BlockSpec((D, 3 * D), lambda b: (0, 0)),
            pl.BlockSpec((D, D), lambda b: (0, 0)),
            pl.BlockSpec((D, D_FF), lambda b: (0, 0)),
            pl.BlockSpec((D_FF, D), lambda b: (0, 0)),
        ],
        out_specs=pl.BlockSpec((1, NT, D), lambda b: (b, 0, 0)),
        out_shape=jax.ShapeDtypeStruct((B, NT, D), jnp.float32),
    )(xp, W_patch, b_patch, clspos, W_qkv, W_o, W_fc1, W_fc2)


DD_R = 392


def _dedup_kernel(col_ref, row_ref, out_ref):
    jb = pl.program_id(0)
    col = col_ref[...]
    row = row_ref[...]
    eq = col == row
    ig = jb * DD_R + lax.broadcasted_iota(jnp.int32, (DD_R, BN), 0)
    jc = lax.broadcasted_iota(jnp.int32, (DD_R, BN), 1)
    dup = jnp.any(eq & (jc > ig), axis=1, keepdims=True)
    out_ref[...] = jnp.where(dup, DUP_SLOT, col)


def _dedup(write_idx):
    return pl.pallas_call(
        _dedup_kernel,
        grid=(BN // DD_R,),
        in_specs=[
            pl.BlockSpec((DD_R, 1), lambda j: (j, 0)),
            pl.BlockSpec((1, BN), lambda j: (0, 0)),
        ],
        out_specs=pl.BlockSpec((DD_R, 1), lambda j: (j, 0)),
        out_shape=jax.ShapeDtypeStruct((BN, 1), jnp.int32),
    )(write_idx.reshape(BN, 1), write_idx.reshape(1, BN))


@functools.cache
def _sc_mesh():
    return plsc.VectorSubcoreMesh(core_axis_name="c", subcore_axis_name="s")


def _sc_scatter_body(mem_ref, flat_ref, idx_ref, idxv, rowsv, sem):
    w = lax.axis_index("s") * 2 + lax.axis_index("c")
    pltpu.sync_copy(idx_ref.at[w], idxv)
    pltpu.sync_copy(flat_ref.at[pl.ds(w * UPW, UPW)], rowsv)
    pltpu.async_copy(rowsv, mem_ref.at[idxv], sem).wait()


def _sc_scatter(mem_state, flat_pad, idx2d):
    pl.kernel(
        _sc_scatter_body,
        out_type=(),
        mesh=_sc_mesh(),
        scratch_types=[
            pltpu.VMEM((UPW,), jnp.int32),
            pltpu.VMEM((UPW, D), jnp.float32),
            pltpu.SemaphoreType.DMA,
        ],
    )(mem_state, flat_pad, idx2d)


GC = 56


def _sc_gather_body(mem_ref, nn_ref, out_ref, idxv, rowsv, sem):
    w = lax.axis_index("s") * 2 + lax.axis_index("c")
    base = w * QPW * 8
    pltpu.sync_copy(nn_ref.at[pl.ds(base, QPW * 8)], idxv)
    for c in range(QPW * 8 // GC):
        pltpu.async_copy(mem_ref.at[idxv.at[pl.ds(c * GC, GC)]], rowsv, sem).wait()
        pltpu.sync_copy(rowsv, out_ref.at[pl.ds(base + c * GC, GC)])


def _sc_gather(mem2, nn_flat):
    return pl.kernel(
        _sc_gather_body,
        out_type=jax.ShapeDtypeStruct((BN * 8, D), jnp.float32),
        mesh=_sc_mesh(),
        scratch_types=[
            pltpu.VMEM((QPW * 8,), jnp.int32),
            pltpu.VMEM((GC, D), jnp.float32),
            pltpu.SemaphoreType.DMA,
        ],
    )(mem2, nn_flat)


def _batchsum_kernel(g_ref, out_ref):
    out_ref[0, 0] = jnp.sum(g_ref[0], axis=0)


def _batchsum(g3):
    out = pl.pallas_call(
        _batchsum_kernel,
        grid=(B,),
        in_specs=[pl.BlockSpec((1, N * 8, D), lambda b: (b, 0, 0))],
        out_specs=pl.BlockSpec((1, 1, D), lambda b: (b, 0, 0)),
        out_shape=jax.ShapeDtypeStruct((B, 1, D), jnp.float32),
    )(g3)
    return out.reshape(B, D)


def _simtopk_kernel(flat_ref, mem_ref, idx_ref, qn_s, bv_s, bi_s):
    j = pl.program_id(0)

    @pl.when(j == 0)
    def _init():
        f = flat_ref[...]
        nrm = jnp.sqrt(jnp.sum(f * f, axis=1, keepdims=True)) + 1e-6
        qn_s[...] = f / nrm
        bv_s[...] = jnp.full((BN, 8), NEG, jnp.float32)
        bi_s[...] = jnp.full((BN, 8), BIGI, jnp.int32)

    mem = mem_ref[...]
    mnrm = jnp.sqrt(jnp.sum(mem * mem, axis=1, keepdims=True)) + 1e-6
    mn = mem / mnrm
    sim = lax.dot_general(qn_s[...], mn, (((1,), (1,)), ((), ())),
                          preferred_element_type=jnp.float32)
    colidx = j * MC + lax.broadcasted_iota(jnp.int32, (BN, MC), 1)
    bv = bv_s[...]
    bi = bi_s[...]
    nv, ni = [], []
    for _ in range(K):
        m = jnp.maximum(jnp.max(sim, axis=1, keepdims=True),
                        jnp.max(bv, axis=1, keepdims=True))
        i1 = jnp.min(jnp.where(sim == m, colidx, BIGI), axis=1, keepdims=True)
        i2 = jnp.min(jnp.where(bv == m, bi, BIGI), axis=1, keepdims=True)
        ii = jnp.minimum(i1, i2)
        nv.append(m)
        ni.append(ii)
        sim = jnp.where(colidx == ii, NEG, sim)
        bv = jnp.where(bi == ii, NEG, bv)
    pad_v = jnp.full((BN, 8 - K), NEG, jnp.float32)
    pad_i = jnp.full((BN, 8 - K), BIGI, jnp.int32)
    bv_s[...] = jnp.concatenate(nv + [pad_v], axis=1)
    bi_s[...] = jnp.concatenate(ni + [pad_i], axis=1)

    @pl.when(j == pl.num_programs(0) - 1)
    def _fin():
        lane = lax.broadcasted_iota(jnp.int32, (BN, 8), 1)
        idx_ref[...] = jnp.where(lane < K, bi_s[...], ZERO_SLOT)


def _simtopk(flat, mem2):
    nsteps = CAP // MC
    return pl.pallas_call(
        _simtopk_kernel,
        grid=(nsteps,),
        in_specs=[
            pl.BlockSpec((BN, D), lambda j: (0, 0)),
            pl.BlockSpec((MC, D), lambda j: (j, 0)),
        ],
        out_specs=pl.BlockSpec((BN, 8), lambda j: (0, 0)),
        out_shape=jax.ShapeDtypeStruct((BN, 8), jnp.int32),
        scratch_shapes=[
            pltpu.VMEM((BN, D), jnp.float32),
            pltpu.VMEM((BN, 8), jnp.float32),
            pltpu.VMEM((BN, 8), jnp.int32),
        ],
    )(flat, mem2)


def _combine_kernel(flat_ref, cls_ref, nsum_ref, cm_ref, loss_ref):
    fm = jnp.mean(flat_ref[...], axis=1)
    cm = REMAIN * fm + (1.0 - REMAIN) / (K * N) * nsum_ref[...]
    cs = cls_ref[...]
    num = jnp.sum(cs * cm, axis=-1)
    den = jnp.sqrt(jnp.sum(cs * cs, axis=-1)) * jnp.sqrt(jnp.sum(cm * cm, axis=-1)) + 1e-8
    loss = jnp.mean(1.0 - num / den)
    cm_ref[...] = cm
    loss_ref[...] = jnp.full((1, 1), loss, jnp.float32)


def _combine(flat3, cls_signal, neigh_sum):
    cm, loss = pl.pallas_call(
        _combine_kernel,
        out_shape=(
            jax.ShapeDtypeStruct((B, D), jnp.float32),
            jax.ShapeDtypeStruct((1, 1), jnp.float32),
        ),
    )(flat3, cls_signal, neigh_sum)
    return cm, loss.reshape(())


def kernel(x, W_patch, b_patch, cls_tok, pos_emb, W_qkv, W_o, W_fc1, W_fc2, w_score, memory, write_idx):
    Bn = x.shape[0]
    xp = x.reshape(Bn, C, HW // P, P, HW // P, P)
    xp = xp.transpose(0, 2, 4, 1, 3, 5).reshape(Bn, N, C * P * P)
    clspos = jnp.concatenate(
        [cls_tok[0] + pos_emb[0, :1], pos_emb[0, 1:],
         jnp.zeros((NT - 1 - N, D), jnp.float32)], axis=0)
    tok_out = _encoder(xp, W_patch, b_patch.reshape(1, D), clspos,
                       W_qkv, W_o, W_fc1, W_fc2)
    cls_signal = tok_out[:, 0]
    flat = tok_out[:, 1:N + 1].reshape(Bn * N, D)
    safe_idx = _dedup(write_idx).reshape(BN)
    idx2d = jnp.concatenate(
        [safe_idx, jnp.full((UPD_PAD - BN,), DUP_SLOT, jnp.int32)]).reshape(NSC, UPW)
    flat_pad = jnp.concatenate([flat, jnp.zeros((UPD_PAD - BN, D), jnp.float32)], axis=0)
    mem_pad = jnp.concatenate([memory, jnp.zeros((CAP_PAD - CAP, D), jnp.float32)], axis=0)
    mem_state = jax.new_ref(mem_pad)
    _sc_scatter(mem_state, flat_pad, idx2d)
    mem2 = mem_state[...]
    nn8 = _simtopk(flat, mem2)
    g = _sc_gather(mem2, nn8.reshape(BN * 8))
    neigh_sum = _batchsum(g.reshape(B, N * 8, D))
    return _combine(flat.reshape(Bn, N, D), cls_signal, neigh_sum)


# speedup vs baseline: 1.4404x; 1.4404x over previous
import functools

import jax
import jax.numpy as jnp
from jax import lax
from jax.experimental import pallas as pl
from jax.experimental.pallas import tpu as pltpu
from jax.experimental.pallas import tpu_sc as plsc

B, C, HW, P = 8, 3, 224, 16
N = (HW // P) ** 2
D = 768
D_FF = 3072
H = 12
CAP = 10000
K = 5
REMAIN = 0.1
BN = B * N

MC = 2000
NEG = -3e38
BIGI = 2**30

CAP_PAD = 10016
DUP_SLOT = 10008
ZERO_SLOT = 10012
UPD_PAD = 1792
NSC = 32
UPW = UPD_PAD // NSC
QPW = BN // NSC


def _ln(t):
    m = jnp.mean(t, axis=-1, keepdims=True)
    v = jnp.var(t, axis=-1, keepdims=True)
    return (t - m) / jnp.sqrt(v + 1e-6)


NT = 200


def _encoder_kernel(xp_ref, wp_ref, bp_ref, cp_ref, wqkv_ref, wo_ref,
                    wfc1_ref, wfc2_ref, tok_ref):
    xp = xp_ref[0]
    tokp = jnp.dot(xp, wp_ref[...], preferred_element_type=jnp.float32) + bp_ref[...]
    tok = jnp.concatenate(
        [jnp.zeros((1, D), jnp.float32), tokp, jnp.zeros((NT - 1 - N, D), jnp.float32)],
        axis=0) + cp_ref[...]
    h = _ln(tok)
    qkv = jnp.dot(h, wqkv_ref[...], preferred_element_type=jnp.float32)
    dh = D // H
    scale = 1.0 / jnp.sqrt(jnp.float32(dh))
    kmask = jnp.where(lax.broadcasted_iota(jnp.int32, (1, NT), 1) < N + 1,
                      0.0, NEG).astype(jnp.float32)
    outs = []
    for hh in range(H):
        qh = qkv[:, hh * dh:(hh + 1) * dh]
        kh = qkv[:, D + hh * dh:D + (hh + 1) * dh]
        vh = qkv[:, 2 * D + hh * dh:2 * D + (hh + 1) * dh]
        s = lax.dot_general(qh, kh, (((1,), (1,)), ((), ())),
                            preferred_element_type=jnp.float32) * scale
        s = s + kmask
        s = s - jnp.max(s, axis=1, keepdims=True)
        e = jnp.exp(s)
        a = e / jnp.sum(e, axis=1, keepdims=True)
        outs.append(jnp.dot(a, vh, preferred_element_type=jnp.float32))
    o = jnp.concatenate(outs, axis=1)
    tok = tok + jnp.dot(o, wo_ref[...], preferred_element_type=jnp.float32)
    h2 = jnp.dot(_ln(tok), wfc1_ref[...], preferred_element_type=jnp.float32)
    tok = tok + jnp.dot(jax.nn.gelu(h2), wfc2_ref[...],
                        preferred_element_type=jnp.float32)
    tok_ref[0] = tok


def _encoder(xp, W_patch, b_patch, clspos, W_qkv, W_o, W_fc1, W_fc2):
    CPP = C * P * P
    return pl.pallas_call(
        _encoder_kernel,
        grid=(B,),
        in_specs=[
            pl.BlockSpec((1, N, CPP), lambda b: (b, 0, 0)),
            pl.BlockSpec((CPP, D), lambda b: (0, 0)),
            pl.BlockSpec((1, D), lambda b: (0, 0)),
            pl.BlockSpec((NT, D), lambda b: (0, 0)),
            pl.BlockSpec((D, 3 * D), lambda b: (0, 0)),
            pl.BlockSpec((D, D), lambda b: (0, 0)),
            pl.BlockSpec((D, D_FF), lambda b: (0, 0)),
            pl.BlockSpec((D_FF, D), lambda b: (0, 0)),
        ],
        out_specs=pl.BlockSpec((1, NT, D), lambda b: (b, 0, 0)),
        out_shape=jax.ShapeDtypeStruct((B, NT, D), jnp.float32),
    )(xp, W_patch, b_patch, clspos, W_qkv, W_o, W_fc1, W_fc2)


DD_R = 392


def _dedup_kernel(col_ref, row_ref, out_ref):
    jb = pl.program_id(0)
    col = col_ref[...]
    row = row_ref[...]
    eq = col == row
    ig = jb * DD_R + lax.broadcasted_iota(jnp.int32, (DD_R, BN), 0)
    jc = lax.broadcasted_iota(jnp.int32, (DD_R, BN), 1)
    dup = jnp.any(eq & (jc > ig), axis=1, keepdims=True)
    out_ref[...] = jnp.where(dup, DUP_SLOT, col)


def _dedup(write_idx):
    return pl.pallas_call(
        _dedup_kernel,
        grid=(BN // DD_R,),
        in_specs=[
            pl.BlockSpec((DD_R, 1), lambda j: (j, 0)),
            pl.BlockSpec((1, BN), lambda j: (0, 0)),
        ],
        out_specs=pl.BlockSpec((DD_R, 1), lambda j: (j, 0)),
        out_shape=jax.ShapeDtypeStruct((BN, 1), jnp.int32),
    )(write_idx.reshape(BN, 1), write_idx.reshape(1, BN))


@functools.cache
def _sc_mesh():
    return plsc.VectorSubcoreMesh(core_axis_name="c", subcore_axis_name="s")


def _sc_scatter_body(mem_ref, flat_ref, idx_ref, idxv, rowsv, sem):
    w = lax.axis_index("s") * 2 + lax.axis_index("c")
    pltpu.sync_copy(idx_ref.at[w], idxv)
    pltpu.sync_copy(flat_ref.at[pl.ds(w * UPW, UPW)], rowsv)
    pltpu.async_copy(rowsv, mem_ref.at[idxv], sem).wait()


def _sc_scatter(mem_state, flat_pad, idx2d):
    pl.kernel(
        _sc_scatter_body,
        out_type=(),
        mesh=_sc_mesh(),
        scratch_types=[
            pltpu.VMEM((UPW,), jnp.int32),
            pltpu.VMEM((UPW, D), jnp.float32),
            pltpu.SemaphoreType.DMA,
        ],
    )(mem_state, flat_pad, idx2d)


GPW = 248
GCH = ((0, 56), (56, 56), (112, 56), (168, 56), (224, 24))
LANES = 16


def _gacc(acc, buf, nrows):
    def body(r, _):
        for jj in range(D // LANES):
            sl = pl.ds(jj * LANES, LANES)
            acc[sl] = acc[sl] + buf[r, sl]
        return ()
    lax.fori_loop(0, nrows, body, (), unroll=False)


def _sc_gather_body(mem_ref, nn_ref, out_ref, idxv, buf0, buf1, acc, sem0, sem1):
    w = lax.axis_index("s") * 2 + lax.axis_index("c")
    pltpu.sync_copy(nn_ref.at[pl.ds(w * GPW, GPW)], idxv)
    for jj in range(D // LANES):
        acc[pl.ds(jj * LANES, LANES)] = jnp.zeros((LANES,), jnp.float32)
    bufs = (buf0, buf1)
    sems = (sem0, sem1)
    cps = []
    for c, (off, nr) in enumerate(GCH):
        cps.append(pltpu.async_copy(
            mem_ref.at[idxv.at[pl.ds(off, nr)]],
            bufs[c % 2].at[pl.ds(0, nr)], sems[c % 2]))
        if c >= 1:
            cps[c - 1].wait()
            _gacc(acc, bufs[(c - 1) % 2], GCH[c - 1][1])
    cps[-1].wait()
    _gacc(acc, bufs[(len(GCH) - 1) % 2], GCH[-1][1])
    pltpu.sync_copy(acc, out_ref.at[w])


def _sc_gather(mem2, nn5_flat):
    return pl.kernel(
        _sc_gather_body,
        out_type=jax.ShapeDtypeStruct((NSC, D), jnp.float32),
        mesh=_sc_mesh(),
        scratch_types=[
            pltpu.VMEM((GPW,), jnp.int32),
            pltpu.VMEM((56, D), jnp.float32),
            pltpu.VMEM((56, D), jnp.float32),
            pltpu.VMEM((D,), jnp.float32),
            pltpu.SemaphoreType.DMA,
            pltpu.SemaphoreType.DMA,
        ],
    )(mem2, nn5_flat)


def _simtopk_kernel(flat_ref, mem_ref, idx_ref, qn_s, bv_s, bi_s):
    j = pl.program_id(0)

    @pl.when(j == 0)
    def _init():
        f = flat_ref[...]
        nrm = jnp.sqrt(jnp.sum(f * f, axis=1, keepdims=True)) + 1e-6
        qn_s[...] = f / nrm
        bv_s[...] = jnp.full((BN, 8), NEG, jnp.float32)
        bi_s[...] = jnp.full((BN, 8), BIGI, jnp.int32)

    mem = mem_ref[...]
    mnrm = jnp.sqrt(jnp.sum(mem * mem, axis=1, keepdims=True)) + 1e-6
    mn = mem / mnrm
    sim = lax.dot_general(qn_s[...], mn, (((1,), (1,)), ((), ())),
                          preferred_element_type=jnp.float32)
    colidx = j * MC + lax.broadcasted_iota(jnp.int32, (BN, MC), 1)
    bv = bv_s[...]
    bi = bi_s[...]
    nv, ni = [], []
    for _ in range(K):
        m = jnp.maximum(jnp.max(sim, axis=1, keepdims=True),
                        jnp.max(bv, axis=1, keepdims=True))
        i1 = jnp.min(jnp.where(sim == m, colidx, BIGI), axis=1, keepdims=True)
        i2 = jnp.min(jnp.where(bv == m, bi, BIGI), axis=1, keepdims=True)
        ii = jnp.minimum(i1, i2)
        nv.append(m)
        ni.append(ii)
        sim = jnp.where(colidx == ii, NEG, sim)
        bv = jnp.where(bi == ii, NEG, bv)
    pad_v = jnp.full((BN, 8 - K), NEG, jnp.float32)
    pad_i = jnp.full((BN, 8 - K), BIGI, jnp.int32)
    bv_s[...] = jnp.concatenate(nv + [pad_v], axis=1)
    bi_s[...] = jnp.concatenate(ni + [pad_i], axis=1)

    @pl.when(j == pl.num_programs(0) - 1)
    def _fin():
        lane = lax.broadcasted_iota(jnp.int32, (BN, 8), 1)
        idx_ref[...] = jnp.where(lane < K, bi_s[...], ZERO_SLOT)


def _simtopk(flat, mem2):
    nsteps = CAP // MC
    return pl.pallas_call(
        _simtopk_kernel,
        grid=(nsteps,),
        in_specs=[
            pl.BlockSpec((BN, D), lambda j: (0, 0)),
            pl.BlockSpec((MC, D), lambda j: (j, 0)),
        ],
        out_specs=pl.BlockSpec((BN, 8), lambda j: (0, 0)),
        out_shape=jax.ShapeDtypeStruct((BN, 8), jnp.int32),
        scratch_shapes=[
            pltpu.VMEM((BN, D), jnp.float32),
            pltpu.VMEM((BN, 8), jnp.float32),
            pltpu.VMEM((BN, 8), jnp.int32),
        ],
    )(flat, mem2)


def _combine_kernel(flat_ref, cls_ref, nsum_ref, cm_ref, loss_ref):
    fm = jnp.mean(flat_ref[...], axis=1)
    nsum = jnp.sum(nsum_ref[...], axis=1)
    cm = REMAIN * fm + (1.0 - REMAIN) / (K * N) * nsum
    cs = cls_ref[...]
    num = jnp.sum(cs * cm, axis=-1)
    den = jnp.sqrt(jnp.sum(cs * cs, axis=-1)) * jnp.sqrt(jnp.sum(cm * cm, axis=-1)) + 1e-8
    loss = jnp.mean(1.0 - num / den)
    cm_ref[...] = cm
    loss_ref[...] = jnp.full((1, 1), loss, jnp.float32)


def _combine(flat3, cls_signal, neigh_sum):
    cm, loss = pl.pallas_call(
        _combine_kernel,
        out_shape=(
            jax.ShapeDtypeStruct((B, D), jnp.float32),
            jax.ShapeDtypeStruct((1, 1), jnp.float32),
        ),
    )(flat3, cls_signal, neigh_sum)
    return cm, loss.reshape(())


def kernel(x, W_patch, b_patch, cls_tok, pos_emb, W_qkv, W_o, W_fc1, W_fc2, w_score, memory, write_idx):
    Bn = x.shape[0]
    xp = x.reshape(Bn, C, HW // P, P, HW // P, P)
    xp = xp.transpose(0, 2, 4, 1, 3, 5).reshape(Bn, N, C * P * P)
    clspos = jnp.concatenate(
        [cls_tok[0] + pos_emb[0, :1], pos_emb[0, 1:],
         jnp.zeros((NT - 1 - N, D), jnp.float32)], axis=0)
    tok_out = _encoder(xp, W_patch, b_patch.reshape(1, D), clspos,
                       W_qkv, W_o, W_fc1, W_fc2)
    cls_signal = tok_out[:, 0]
    flat = tok_out[:, 1:N + 1].reshape(Bn * N, D)
    safe_idx = _dedup(write_idx).reshape(BN)
    idx2d = jnp.concatenate(
        [safe_idx, jnp.full((UPD_PAD - BN,), DUP_SLOT, jnp.int32)]).reshape(NSC, UPW)
    flat_pad = jnp.concatenate([flat, jnp.zeros((UPD_PAD - BN, D), jnp.float32)], axis=0)
    mem_pad = jnp.concatenate([memory, jnp.zeros((CAP_PAD - CAP, D), jnp.float32)], axis=0)
    mem_state = jax.new_ref(mem_pad)
    _sc_scatter(mem_state, flat_pad, idx2d)
    mem2 = mem_state[...]
    nn8 = _simtopk(flat, mem2)
    nn5 = jnp.concatenate(
        [nn8[:, :K].reshape(NSC, QPW * K),
         jnp.full((NSC, GPW - QPW * K), ZERO_SLOT, jnp.int32)], axis=1)
    partials = _sc_gather(mem2, nn5.reshape(NSC * GPW))
    return _combine(flat.reshape(Bn, N, D), cls_signal, partials.reshape(B, 4, D))
